# baseline (speedup 1.0000x reference)
import functools
import math

import jax
import jax.numpy as jnp
import numpy as np
from jax import lax
from jax.experimental import pallas as pl
from jax.experimental.pallas import tpu as pltpu
from jax.experimental.pallas import tpu_sc as plsc

B = 4
NW = 32
L = 16


def _mm(a, w):
    return lax.dot_general(a, w, (((1,), (1,)), ((), ())),
                           preferred_element_type=jnp.float32)


def _stn_body(x_ref, w1, b1, w2, b2, w3, b3, fw1, fb1, fw2, fb2, fw3, fb3,
              t9_ref, xp_ref):
    x = x_ref[0]
    n = x.shape[1]
    nch = 4
    ch = n // nch
    mx = None
    for ci in range(nch):
        xc = x[:, ci * ch:(ci + 1) * ch]
        h = jnp.maximum(jnp.dot(w1[...], xc, preferred_element_type=jnp.float32) + b1[...], 0.0)
        h = jnp.maximum(jnp.dot(w2[...], h, preferred_element_type=jnp.float32) + b2[...], 0.0)
        h = jnp.maximum(jnp.dot(w3[...], h, preferred_element_type=jnp.float32) + b3[...], 0.0)
        hm = jnp.max(h, axis=1, keepdims=True)
        mx = hm if mx is None else jnp.maximum(mx, hm)
    h = jnp.maximum(jnp.dot(fw1[...], mx, preferred_element_type=jnp.float32) + fb1[...], 0.0)
    h = jnp.maximum(jnp.dot(fw2[...], h, preferred_element_type=jnp.float32) + fb2[...], 0.0)
    t9 = jnp.dot(fw3[...], h, preferred_element_type=jnp.float32) + fb3[...]
    iden = (lax.broadcasted_iota(jnp.int32, (9, 1), 0) % 4 == 0).astype(jnp.float32)
    t9 = t9 + iden
    t9_ref[0] = t9
    tb = t9.astype(jnp.bfloat16).astype(jnp.float32)
    xb = x.astype(jnp.bfloat16).astype(jnp.float32)
    rows = []
    for j in range(3):
        r = (tb[j:j + 1, :] * xb[0:1, :]
             + tb[3 + j:4 + j, :] * xb[1:2, :]
             + tb[6 + j:7 + j, :] * xb[2:3, :])
        rows.append(r)
    xp_ref[0] = jnp.concatenate(rows, axis=0)


def _stn_call(xyz, params):
    n = xyz.shape[2]
    cb = lambda a: a.reshape(-1, 1)
    args = [xyz,
            params['stn_conv1_w'], cb(params['stn_conv1_b']),
            params['stn_conv2_w'], cb(params['stn_conv2_b']),
            params['stn_conv3_w'], cb(params['stn_conv3_b']),
            params['stn_fc1_w'], cb(params['stn_fc1_b']),
            params['stn_fc2_w'], cb(params['stn_fc2_b']),
            params['stn_fc3_w'], cb(params['stn_fc3_b'])]
    in_specs = [pl.BlockSpec((1, 3, n), lambda b: (b, 0, 0))]
    for a in args[1:]:
        in_specs.append(pl.BlockSpec(a.shape, lambda b, nd=a.ndim: (0,) * nd))
    t9, xp = pl.pallas_call(
        _stn_body,
        grid=(B,),
        in_specs=in_specs,
        out_specs=[pl.BlockSpec((1, 9, 1), lambda b: (b, 0, 0)),
                   pl.BlockSpec((1, 3, n), lambda b: (b, 0, 0))],
        out_shape=[jax.ShapeDtypeStruct((B, 9, 1), jnp.float32),
                   jax.ShapeDtypeStruct((B, 3, n), jnp.float32)],
    )(*args)
    return t9.reshape(B, 3, 3), xp


def _fps_body(n, s, xyz_v, idx_ref):
    ln = min(n, 128)
    rn = n // ln
    lin = lax.broadcasted_iota(jnp.int32, (rn, ln), 0) * ln + \
        lax.broadcasted_iota(jnp.int32, (rn, ln), 1)
    xs = [[xyz_v[b, c, :].reshape(rn, ln) for c in range(3)] for b in range(B)]

    def body(i, carry):
        fars, dists = carry
        new_fars = []
        new_dists = []
        for b in range(B):
            f = fars[b]
            idx_ref[b, i] = f + b * n
            fl = lax.rem(f, ln)
            lanes = lax.iota(jnp.int32, ln)
            if rn == 1:
                row = xyz_v[b, :, :]
            else:
                fb = pl.multiple_of(f - fl, ln)
                row = xyz_v[b, :, pl.ds(fb, ln)]
            sel = jnp.where(lanes[None, :] == fl, row, 0.0)
            cx = jnp.sum(sel[0, :])
            cy = jnp.sum(sel[1, :])
            cz = jnp.sum(sel[2, :])
            d = ((xs[b][0] - cx) ** 2 + (xs[b][1] - cy) ** 2
                 + (xs[b][2] - cz) ** 2)
            d = jnp.minimum(dists[b], d)
            m = jnp.max(d)
            far = jnp.min(jnp.where(d == m, lin, n))
            new_fars.append(far)
            new_dists.append(d)
        return tuple(new_fars), tuple(new_dists)

    fars0 = tuple(jnp.int32(0) for _ in range(B))
    dists0 = tuple(jnp.full((rn, ln), 1e10, jnp.float32) for _ in range(B))
    lax.fori_loop(0, s, body, (fars0, dists0))


def _fps_call(l_xyz, s):
    n = l_xyz.shape[2]
    return pl.pallas_call(
        functools.partial(_fps_body, n, s),
        out_specs=pl.BlockSpec(memory_space=pltpu.SMEM),
        out_shape=jax.ShapeDtypeStruct((B, s), jnp.int32),
    )(l_xyz)


def _sqdist_body(nx_ref, xt_ref, out_ref):
    nx = nx_ref[0]
    xt = xt_ref[0]
    sn = jnp.sum(nx * nx, axis=1, keepdims=True)
    xn = jnp.sum(xt * xt, axis=0, keepdims=True)
    mm = jnp.dot(nx, xt, preferred_element_type=jnp.float32)
    out_ref[...] = sn + xn - 2.0 * mm


def _sqdist_call(new_xyz, l_xyz):
    s = new_xyz.shape[1]
    n = l_xyz.shape[2]
    sblk = min(s, 256)
    grid = (B, s // sblk)
    return pl.pallas_call(
        _sqdist_body,
        grid=grid,
        in_specs=[pl.BlockSpec((1, sblk, 3), lambda b, i: (b, i, 0)),
                  pl.BlockSpec((1, 3, n), lambda b, i: (b, 0, 0))],
        out_specs=pl.BlockSpec((sblk, n), lambda b, i: (b * (s // sblk) + i, 0)),
        out_shape=jax.ShapeDtypeStruct((B * s, n), jnp.float32),
    )(new_xyz, l_xyz)


def _ballq_call(dists, r2, s, n, nsample=32):
    r = dists.shape[0]
    r_w = r // NW
    nchunk = n // L
    mesh = plsc.VectorSubcoreMesh(core_axis_name="c", subcore_axis_name="s")
    log2s = int(math.log2(s))

    def body(d_hbm, out_hbm, d_v, o_v):
        wid = lax.axis_index("s") * 2 + lax.axis_index("c")

        def row_step(j, _):
            row = wid * r_w + j
            off = (row >> log2s) * n
            pltpu.sync_copy(d_hbm.at[row], d_v)
            lanes = lax.iota(jnp.int32, L)

            big = jnp.int32(2 ** 30)

            def chunk_step(ck, carry):
                cnt_v, fv = carry
                d = d_v[pl.ds(ck * L, L)]
                m = d <= r2

                def hit_branch(carry):
                    cnt_v, fv = carry
                    keys = jnp.where(m, lanes, 2 * L)
                    pos = lanes + (ck * L + off)
                    _, sv = plsc.sort_key_val(keys, pos)
                    sidx = jnp.minimum(cnt_v, nsample) + lanes
                    plsc.store_scatter(o_v, [sidx], sv)
                    fv = jnp.minimum(fv, jnp.where(m, pos, big))
                    return cnt_v + plsc.all_reduce_population_count(m), fv

                return lax.cond(jnp.any(m), hit_branch, lambda c: c, carry)

            cnt_v, fv = lax.fori_loop(
                0, nchunk, chunk_step,
                (jnp.zeros((L,), jnp.int32), jnp.full((L,), 2 ** 30, jnp.int32)))
            for k in (1, 2, 4, 8):
                fv = jnp.minimum(
                    fv, fv.at[lanes ^ k].get(mode="promise_in_bounds"))
            first = jnp.where(cnt_v > 0, fv, n - 1 + off)
            for half in range(nsample // L):
                ids = lanes + half * L
                cur = o_v[pl.ds(half * L, L)]
                o_v[pl.ds(half * L, L)] = jnp.where(ids < cnt_v, cur, first)
            pltpu.sync_copy(o_v.at[pl.ds(0, nsample)], out_hbm.at[row])
            return 0

        lax.fori_loop(0, r_w, row_step, 0)

    f = pl.kernel(
        body,
        out_type=jax.ShapeDtypeStruct((r, nsample), jnp.int32),
        mesh=mesh,
        compiler_params=pltpu.CompilerParams(use_tc_tiling_on_sc=False,
                                             needs_layout_passes=False),
        scratch_types=[pltpu.VMEM((n,), jnp.float32),
                       pltpu.VMEM((nsample + L,), jnp.int32)],
    )
    return f(dists)


def _sc_gather(table, idx):
    m = idx.shape[0]
    d = table.shape[1]
    m_w = m // NW
    chunk = m_w
    while chunk * d * 4 > 320 * 1024:
        chunk //= 2
    nchunks = m_w // chunk
    mesh = plsc.VectorSubcoreMesh(core_axis_name="c", subcore_axis_name="s")

    def body(table_hbm, idx_hbm, out_hbm, idx_v, rows_v, sem):
        wid = lax.axis_index("s") * 2 + lax.axis_index("c")
        base = wid * m_w

        def step(ci, _):
            off = base + ci * chunk
            pltpu.sync_copy(idx_hbm.at[pl.ds(off, chunk)], idx_v)
            pltpu.async_copy(table_hbm.at[idx_v], rows_v, sem).wait()
            pltpu.sync_copy(rows_v, out_hbm.at[pl.ds(off, chunk)])
            return 0

        lax.fori_loop(0, nchunks, step, 0)

    f = pl.kernel(
        body,
        out_type=jax.ShapeDtypeStruct((m, d), jnp.float32),
        mesh=mesh,
        compiler_params=pltpu.CompilerParams(use_tc_tiling_on_sc=False),
        scratch_types=[pltpu.VMEM((chunk,), jnp.int32),
                       pltpu.VMEM((chunk, d), jnp.float32),
                       pltpu.SemaphoreType.DMA],
    )
    return f(table, idx)


def _samlp_body(nlayer, g_ref, nx_ref, *args):
    wrefs = args[:2 * nlayer]
    out_ref = args[2 * nlayer]
    g = g_ref[...]
    rblk, d = g.shape
    k = 32
    g3 = g.reshape(rblk // k, k, d)
    nx = nx_ref[...]
    gx = g3[:, :, :3] - nx[:, None, :]
    h = jnp.concatenate([gx, g3[:, :, 3:]], axis=2)
    for li in range(nlayer):
        w = wrefs[2 * li][...]
        b = wrefs[2 * li + 1][...]
        h = _mm(h.reshape(h.shape[0] * k, h.shape[2]), w)
        h = jnp.maximum(h + b, 0.0).reshape(rblk // k, k, w.shape[0])
    out_ref[...] = jnp.max(h, axis=1)


def _samlp_call(g, nx_rows, ws):
    m, d = g.shape
    k = 32
    cout = ws[-1][0].shape[0]
    rblk = min(m, 8192)
    grid = (m // rblk,)
    wargs = []
    for (w, b) in ws:
        wp = jnp.pad(w, ((0, 0), (0, d - w.shape[1]))) if w.shape[1] < d else w
        wargs += [wp, b.reshape(1, -1)]
        d = w.shape[0]
    in_specs = [pl.BlockSpec((rblk, g.shape[1]), lambda i: (i, 0)),
                pl.BlockSpec((rblk // k, 3), lambda i: (i, 0))]
    for a in wargs:
        in_specs.append(pl.BlockSpec(a.shape, lambda i, nd=a.ndim: (0,) * nd))
    return pl.pallas_call(
        functools.partial(_samlp_body, len(ws)),
        grid=grid,
        in_specs=in_specs,
        out_specs=pl.BlockSpec((rblk // k, cout), lambda i: (i, 0)),
        out_shape=jax.ShapeDtypeStruct((m // k, cout), jnp.float32),
    )(g, nx_rows, *wargs)


def _top3_body(n2, x1_ref, x2_ref, w_ref, idx_ref):
    bi = pl.program_id(0)
    x1 = x1_ref[0]
    x2 = x2_ref[0]
    sn = jnp.sum(x1 * x1, axis=1, keepdims=True)
    xn = jnp.sum(x2 * x2, axis=0, keepdims=True)
    mm = jnp.dot(x1, x2, preferred_element_type=jnp.float32)
    d = sn + xn - 2.0 * mm
    lanes = lax.broadcasted_iota(jnp.int32, (1, n2), 1)
    vals, idxs = [], []
    for _ in range(3):
        mk = jnp.min(d, axis=1, keepdims=True)
        ik = jnp.min(jnp.where(d == mk, lanes, n2), axis=1, keepdims=True)
        d = jnp.where(lanes == ik, jnp.float32(np.inf), d)
        vals.append(mk)
        idxs.append(ik)
    recips = [1.0 / (v + 1e-8) for v in vals]
    norm = recips[0] + recips[1] + recips[2]
    w_ref[0] = jnp.concatenate([r / norm for r in recips], axis=1)
    idx_ref[0] = jnp.concatenate(idxs, axis=1) + bi * n2


def _top3_call(xyz1_rows, xyz2):
    n1 = xyz1_rows.shape[1]
    n2 = xyz2.shape[2]
    n1blk = min(n1, 1024)
    grid = (B, n1 // n1blk)
    return pl.pallas_call(
        functools.partial(_top3_body, n2),
        grid=grid,
        in_specs=[pl.BlockSpec((1, n1blk, 3), lambda b, i: (b, i, 0)),
                  pl.BlockSpec((1, 3, n2), lambda b, i: (b, 0, 0))],
        out_specs=[pl.BlockSpec((1, n1blk, 3), lambda b, i: (b, i, 0)),
                   pl.BlockSpec((1, n1blk, 3), lambda b, i: (b, i, 0))],
        out_shape=[jax.ShapeDtypeStruct((B, n1, 3), jnp.float32),
                   jax.ShapeDtypeStruct((B, n1, 3), jnp.int32)],
    )(xyz1_rows, xyz2)


def _fpmlp_body(nlayer, has_p1, has_head, g_ref, w_ref, *args):
    pos = 0
    if has_p1:
        p1_ref = args[0]
        pos = 1
    wrefs = args[pos:pos + 2 * nlayer + (4 if has_head else 0)]
    out_ref = args[pos + len(wrefs)]
    g = g_ref[...]
    rb3, dd = g.shape
    rb = rb3 // 3
    w = w_ref[...]
    gw = (g * w).reshape(rb, 3, dd)
    interp = gw[:, 0, :] + gw[:, 1, :] + gw[:, 2, :]
    if has_p1:
        h = jnp.concatenate([p1_ref[...], interp], axis=1)
    else:
        h = interp
    for li in range(nlayer):
        wt = wrefs[2 * li][...]
        b = wrefs[2 * li + 1][...]
        h = jnp.maximum(_mm(h, wt) + b, 0.0)
    if has_head:
        hw1, hb1, hw2, hb2 = [wrefs[2 * nlayer + i][...] for i in range(4)]
        h = jnp.maximum(_mm(h, hw1) + hb1, 0.0)
        z = _mm(h, hw2) + hb2
        zm = jnp.max(z, axis=1, keepdims=True)
        sh = z - zm
        h = sh - jnp.log(jnp.sum(jnp.exp(sh), axis=1, keepdims=True))
    out_ref[...] = h


def _fpmlp_call(g, w_rows, p1_rows, ws, head=None):
    m3, dd = g.shape
    rows = m3 // 3
    rblk = min(rows, 2048)
    grid = (rows // rblk,)
    c1 = p1_rows.shape[1] if p1_rows is not None else 0
    wargs = []
    cin = c1 + dd
    for (wt, b) in ws:
        wp = jnp.pad(wt, ((0, 0), (0, cin - wt.shape[1]))) if wt.shape[1] < cin else wt
        wargs += [wp, b.reshape(1, -1)]
        cin = wt.shape[0]
    cout = ws[-1][0].shape[0]
    if head is not None:
        hw1, hb1, hw2, hb2 = head
        wargs += [hw1, hb1.reshape(1, -1), hw2, hb2.reshape(1, -1)]
        cout = hw2.shape[0]
    args = [g, w_rows] + ([p1_rows] if p1_rows is not None else []) + wargs
    in_specs = [pl.BlockSpec((rblk * 3, dd), lambda i: (i, 0)),
                pl.BlockSpec((rblk * 3, 1), lambda i: (i, 0))]
    if p1_rows is not None:
        in_specs.append(pl.BlockSpec((rblk, c1), lambda i: (i, 0)))
    for a in wargs:
        in_specs.append(pl.BlockSpec(a.shape, lambda i, nd=a.ndim: (0,) * nd))
    return pl.pallas_call(
        functools.partial(_fpmlp_body, len(ws), p1_rows is not None,
                          head is not None),
        grid=grid,
        in_specs=in_specs,
        out_specs=pl.BlockSpec((rblk, cout), lambda i: (i, 0)),
        out_shape=jax.ShapeDtypeStruct((rows, cout), jnp.float32),
    )(*args)


def _pad_cols(a, mult=16):
    c = a.shape[1]
    pc = -c % mult
    if pc:
        a = jnp.pad(a, ((0, 0), (0, pc)))
    return a


def _mlp_params(params, name, nl):
    return [(params[name + '_mlp%d_w' % i], params[name + '_mlp%d_b' % i])
            for i in range(nl)]


def _ballq_tmp(dists, r2, s, n, nsample=32):
    r = dists.shape[0]
    gi = jnp.broadcast_to(jnp.arange(n, dtype=jnp.int32), (r, n))
    gi = jnp.where(dists > r2, n, gi)
    gi = jnp.sort(gi, axis=-1)[:, :nsample]
    first = gi[:, :1]
    gi = jnp.where(gi == n, jnp.broadcast_to(first, gi.shape), gi)
    gi = jnp.minimum(gi, n - 1)
    off = ((jnp.arange(r, dtype=jnp.int32) // s) * n)[:, None]
    return gi + off


def _sa_level(l_xyz, l_pts, s, radius, ws):
    n = l_xyz.shape[2]
    c = l_pts.shape[1]
    table = jnp.concatenate(
        [jnp.transpose(l_xyz, (0, 2, 1)).reshape(B * n, 3),
         jnp.transpose(l_pts, (0, 2, 1)).reshape(B * n, c)], axis=1)
    table = _pad_cols(table)
    fps_idx = _fps_call(l_xyz, s).reshape(-1)
    npad = -fps_idx.shape[0] % (8 * NW)
    fps_idx_p = jnp.pad(fps_idx, (0, npad)) if npad else fps_idx
    new_xyz = _sc_gather(table, fps_idx_p)[:B * s, :3].reshape(B, s, 3)
    dists = _sqdist_call(new_xyz, l_xyz)
    gidx = _ballq_call(dists, radius * radius, s, n)
    g = _sc_gather(table, gidx.reshape(-1))
    feats = _samlp_call(g, new_xyz.reshape(B * s, 3), ws)
    new_l_xyz = jnp.transpose(new_xyz, (0, 2, 1))
    new_pts = jnp.transpose(feats.reshape(B, s, -1), (0, 2, 1))
    return new_l_xyz, new_pts


def _fp_level(xyz1, xyz2, pts1, pts2, ws, head=None):
    n1 = xyz1.shape[2]
    n2 = xyz2.shape[2]
    c2 = pts2.shape[1]
    w3, idx3 = _top3_call(jnp.transpose(xyz1, (0, 2, 1)), xyz2)
    table = _pad_cols(jnp.transpose(pts2, (0, 2, 1)).reshape(B * n2, c2))
    g = _sc_gather(table, idx3.reshape(-1))
    p1_rows = None
    if pts1 is not None:
        p1_rows = jnp.transpose(pts1, (0, 2, 1)).reshape(B * n1, -1)
    out = _fpmlp_call(g, w3.reshape(B * n1 * 3, 1), p1_rows, ws, head)
    return out, n1


def kernel(xyz, input_for_alignment_all_structure, params):
    xyz = xyz.astype(jnp.float32)
    n = xyz.shape[2]
    trans, l0_points = _stn_call(xyz, params)
    l0_xyz = xyz[:, :3, :]

    l1_xyz, l1_points = _sa_level(l0_xyz, l0_points, 1024, 0.1,
                                  _mlp_params(params, 'sa1', 3))
    l2_xyz, l2_points = _sa_level(l1_xyz, l1_points, 256, 0.2,
                                  _mlp_params(params, 'sa2', 3))
    l3_xyz, l3_points = _sa_level(l2_xyz, l2_points, 64, 0.4,
                                  _mlp_params(params, 'sa3', 3))
    l4_xyz, l4_points = _sa_level(l3_xyz, l3_points, 16, 0.8,
                                  _mlp_params(params, 'sa4', 3))

    o, n1 = _fp_level(l3_xyz, l4_xyz, l3_points, l4_points,
                      _mlp_params(params, 'fp4', 2))
    fp4_pts = jnp.transpose(o.reshape(B, n1, -1), (0, 2, 1))
    o, n1 = _fp_level(l2_xyz, l3_xyz, l2_points, fp4_pts,
                      _mlp_params(params, 'fp3', 2))
    fp3_pts = jnp.transpose(o.reshape(B, n1, -1), (0, 2, 1))
    o, n1 = _fp_level(l1_xyz, l2_xyz, l1_points, fp3_pts,
                      _mlp_params(params, 'fp2', 2))
    fp2_pts = jnp.transpose(o.reshape(B, n1, -1), (0, 2, 1))
    head = (params['head_conv1_w'], params['head_conv1_b'],
            params['head_conv2_w'], params['head_conv2_b'])
    o, n1 = _fp_level(l0_xyz, l1_xyz, None, fp2_pts,
                      _mlp_params(params, 'fp1', 3), head=head)
    x = o.reshape(B, n, 7)
    return (x, trans)


# speedup vs baseline: 1.1847x; 1.1847x over previous
import functools
import math

import jax
import jax.numpy as jnp
import numpy as np
from jax import lax
from jax.experimental import pallas as pl
from jax.experimental.pallas import tpu as pltpu
from jax.experimental.pallas import tpu_sc as plsc

B = 4
NW = 32
L = 16


def _mm(a, w):
    return lax.dot_general(a, w, (((1,), (1,)), ((), ())),
                           preferred_element_type=jnp.float32)


def _stn_body(x_ref, w1, b1, w2, b2, w3, b3, fw1, fb1, fw2, fb2, fw3, fb3,
              t9_ref, xp_ref):
    x = x_ref[0]
    n = x.shape[1]
    nch = 4
    ch = n // nch
    mx = None
    for ci in range(nch):
        xc = x[:, ci * ch:(ci + 1) * ch]
        h = jnp.maximum(jnp.dot(w1[...], xc, preferred_element_type=jnp.float32) + b1[...], 0.0)
        h = jnp.maximum(jnp.dot(w2[...], h, preferred_element_type=jnp.float32) + b2[...], 0.0)
        h = jnp.maximum(jnp.dot(w3[...], h, preferred_element_type=jnp.float32) + b3[...], 0.0)
        hm = jnp.max(h, axis=1, keepdims=True)
        mx = hm if mx is None else jnp.maximum(mx, hm)
    h = jnp.maximum(jnp.dot(fw1[...], mx, preferred_element_type=jnp.float32) + fb1[...], 0.0)
    h = jnp.maximum(jnp.dot(fw2[...], h, preferred_element_type=jnp.float32) + fb2[...], 0.0)
    t9 = jnp.dot(fw3[...], h, preferred_element_type=jnp.float32) + fb3[...]
    iden = (lax.broadcasted_iota(jnp.int32, (9, 1), 0) % 4 == 0).astype(jnp.float32)
    t9 = t9 + iden
    t9_ref[0] = t9
    tb = t9.astype(jnp.bfloat16).astype(jnp.float32)
    xb = x.astype(jnp.bfloat16).astype(jnp.float32)
    rows = []
    for j in range(3):
        r = (tb[j:j + 1, :] * xb[0:1, :]
             + tb[3 + j:4 + j, :] * xb[1:2, :]
             + tb[6 + j:7 + j, :] * xb[2:3, :])
        rows.append(r)
    xp_ref[0] = jnp.concatenate(rows, axis=0)


def _stn_call(xyz, params):
    n = xyz.shape[2]
    cb = lambda a: a.reshape(-1, 1)
    args = [xyz,
            params['stn_conv1_w'], cb(params['stn_conv1_b']),
            params['stn_conv2_w'], cb(params['stn_conv2_b']),
            params['stn_conv3_w'], cb(params['stn_conv3_b']),
            params['stn_fc1_w'], cb(params['stn_fc1_b']),
            params['stn_fc2_w'], cb(params['stn_fc2_b']),
            params['stn_fc3_w'], cb(params['stn_fc3_b'])]
    in_specs = [pl.BlockSpec((1, 3, n), lambda b: (b, 0, 0))]
    for a in args[1:]:
        in_specs.append(pl.BlockSpec(a.shape, lambda b, nd=a.ndim: (0,) * nd))
    t9, xp = pl.pallas_call(
        _stn_body,
        grid=(B,),
        in_specs=in_specs,
        out_specs=[pl.BlockSpec((1, 9, 1), lambda b: (b, 0, 0)),
                   pl.BlockSpec((1, 3, n), lambda b: (b, 0, 0))],
        out_shape=[jax.ShapeDtypeStruct((B, 9, 1), jnp.float32),
                   jax.ShapeDtypeStruct((B, 3, n), jnp.float32)],
    )(*args)
    return t9.reshape(B, 3, 3), xp


def _fps_body(n, s, xyz_v, idx_ref):
    ln = min(n, 128)
    rn = n // ln
    lin = lax.broadcasted_iota(jnp.int32, (rn, ln), 0) * ln + \
        lax.broadcasted_iota(jnp.int32, (rn, ln), 1)
    xs = [[xyz_v[b, c, :].reshape(rn, ln) for c in range(3)] for b in range(B)]

    def body(i, carry):
        fars, dists = carry
        new_fars = []
        new_dists = []
        for b in range(B):
            f = fars[b]
            idx_ref[b, i] = f + b * n
            fl = lax.rem(f, ln)
            lanes = lax.iota(jnp.int32, ln)
            if rn == 1:
                row = xyz_v[b, :, :]
            else:
                fb = pl.multiple_of(f - fl, ln)
                row = xyz_v[b, :, pl.ds(fb, ln)]
            sel = jnp.where(lanes[None, :] == fl, row, 0.0)
            cx = jnp.sum(sel[0, :])
            cy = jnp.sum(sel[1, :])
            cz = jnp.sum(sel[2, :])
            d = ((xs[b][0] - cx) ** 2 + (xs[b][1] - cy) ** 2
                 + (xs[b][2] - cz) ** 2)
            d = jnp.minimum(dists[b], d)
            m = jnp.max(d)
            far = jnp.min(jnp.where(d == m, lin, n))
            new_fars.append(far)
            new_dists.append(d)
        return tuple(new_fars), tuple(new_dists)

    fars0 = tuple(jnp.int32(0) for _ in range(B))
    dists0 = tuple(jnp.full((rn, ln), 1e10, jnp.float32) for _ in range(B))
    lax.fori_loop(0, s, body, (fars0, dists0))


def _fps_call(l_xyz, s):
    n = l_xyz.shape[2]
    return pl.pallas_call(
        functools.partial(_fps_body, n, s),
        out_specs=pl.BlockSpec(memory_space=pltpu.SMEM),
        out_shape=jax.ShapeDtypeStruct((B, s), jnp.int32),
    )(l_xyz)


def _sqdist_body(nx_ref, xt_ref, out_ref):
    nx = nx_ref[0]
    xt = xt_ref[0]
    sn = jnp.sum(nx * nx, axis=1, keepdims=True)
    xn = jnp.sum(xt * xt, axis=0, keepdims=True)
    mm = jnp.dot(nx, xt, preferred_element_type=jnp.float32)
    out_ref[...] = sn + xn - 2.0 * mm


def _sqdist_call(new_xyz, l_xyz):
    s = new_xyz.shape[1]
    n = l_xyz.shape[2]
    sblk = min(s, 256)
    grid = (B, s // sblk)
    return pl.pallas_call(
        _sqdist_body,
        grid=grid,
        in_specs=[pl.BlockSpec((1, sblk, 3), lambda b, i: (b, i, 0)),
                  pl.BlockSpec((1, 3, n), lambda b, i: (b, 0, 0))],
        out_specs=pl.BlockSpec((sblk, n), lambda b, i: (b * (s // sblk) + i, 0)),
        out_shape=jax.ShapeDtypeStruct((B * s, n), jnp.float32),
    )(new_xyz, l_xyz)


def _ballq_call(dists, r2, s, n, nsample=32):
    r = dists.shape[0]
    r_w = r // NW
    nchunk = n // L
    mesh = plsc.VectorSubcoreMesh(core_axis_name="c", subcore_axis_name="s")
    log2s = int(math.log2(s))

    def body(d_hbm, out_hbm, d_v, o_v):
        wid = lax.axis_index("s") * 2 + lax.axis_index("c")

        def row_step(j, _):
            row = wid * r_w + j
            off = (row >> log2s) * n
            pltpu.sync_copy(d_hbm.at[row], d_v)
            lanes = lax.iota(jnp.int32, L)

            big = jnp.int32(2 ** 30)

            def chunk_step(ck, carry):
                cnt_v, fv = carry
                d = d_v[pl.ds(ck * L, L)]
                m = d <= r2
                keys = jnp.where(m, lanes, 2 * L)
                pos = lanes + (ck * L + off)
                _, sv = plsc.sort_key_val(keys, pos)
                sidx = jnp.minimum(cnt_v, nsample) + lanes
                plsc.store_scatter(o_v, [sidx], sv)
                fv = jnp.minimum(fv, jnp.where(m, pos, big))
                return cnt_v + plsc.all_reduce_population_count(m), fv

            cnt_v, fv = lax.fori_loop(
                0, nchunk, chunk_step,
                (jnp.zeros((L,), jnp.int32), jnp.full((L,), 2 ** 30, jnp.int32)),
                unroll=4)
            for k in (1, 2, 4, 8):
                fv = jnp.minimum(
                    fv, fv.at[lanes ^ k].get(mode="promise_in_bounds"))
            first = jnp.where(cnt_v > 0, fv, n - 1 + off)
            for half in range(nsample // L):
                ids = lanes + half * L
                cur = o_v[pl.ds(half * L, L)]
                o_v[pl.ds(half * L, L)] = jnp.where(ids < cnt_v, cur, first)
            pltpu.sync_copy(o_v.at[pl.ds(0, nsample)], out_hbm.at[row])
            return 0

        lax.fori_loop(0, r_w, row_step, 0)

    f = pl.kernel(
        body,
        out_type=jax.ShapeDtypeStruct((r, nsample), jnp.int32),
        mesh=mesh,
        compiler_params=pltpu.CompilerParams(use_tc_tiling_on_sc=False,
                                             needs_layout_passes=False),
        scratch_types=[pltpu.VMEM((n,), jnp.float32),
                       pltpu.VMEM((nsample + L,), jnp.int32)],
    )
    return f(dists)


def _sc_gather(table, idx):
    m = idx.shape[0]
    d = table.shape[1]
    m_w = m // NW
    chunk = m_w
    while chunk * d * 4 > 320 * 1024:
        chunk //= 2
    nchunks = m_w // chunk
    mesh = plsc.VectorSubcoreMesh(core_axis_name="c", subcore_axis_name="s")

    def body(table_hbm, idx_hbm, out_hbm, idx_v, rows_v, sem):
        wid = lax.axis_index("s") * 2 + lax.axis_index("c")
        base = wid * m_w

        def step(ci, _):
            off = base + ci * chunk
            pltpu.sync_copy(idx_hbm.at[pl.ds(off, chunk)], idx_v)
            pltpu.async_copy(table_hbm.at[idx_v], rows_v, sem).wait()
            pltpu.sync_copy(rows_v, out_hbm.at[pl.ds(off, chunk)])
            return 0

        lax.fori_loop(0, nchunks, step, 0)

    f = pl.kernel(
        body,
        out_type=jax.ShapeDtypeStruct((m, d), jnp.float32),
        mesh=mesh,
        compiler_params=pltpu.CompilerParams(use_tc_tiling_on_sc=False),
        scratch_types=[pltpu.VMEM((chunk,), jnp.int32),
                       pltpu.VMEM((chunk, d), jnp.float32),
                       pltpu.SemaphoreType.DMA],
    )
    return f(table, idx)


def _samlp_body(nlayer, g_ref, nx_ref, *args):
    wrefs = args[:2 * nlayer]
    out_ref = args[2 * nlayer]
    g = g_ref[...]
    rblk, d = g.shape
    k = 32
    g3 = g.reshape(rblk // k, k, d)
    nx = nx_ref[...]
    gx = g3[:, :, :3] - nx[:, None, :]
    h = jnp.concatenate([gx, g3[:, :, 3:]], axis=2)
    for li in range(nlayer):
        w = wrefs[2 * li][...]
        b = wrefs[2 * li + 1][...]
        h = _mm(h.reshape(h.shape[0] * k, h.shape[2]), w)
        h = jnp.maximum(h + b, 0.0).reshape(rblk // k, k, w.shape[0])
    out_ref[...] = jnp.max(h, axis=1)


def _samlp_call(g, nx_rows, ws):
    m, d = g.shape
    k = 32
    cout = ws[-1][0].shape[0]
    rblk = min(m, 8192)
    grid = (m // rblk,)
    wargs = []
    for (w, b) in ws:
        wp = jnp.pad(w, ((0, 0), (0, d - w.shape[1]))) if w.shape[1] < d else w
        wargs += [wp, b.reshape(1, -1)]
        d = w.shape[0]
    in_specs = [pl.BlockSpec((rblk, g.shape[1]), lambda i: (i, 0)),
                pl.BlockSpec((rblk // k, 3), lambda i: (i, 0))]
    for a in wargs:
        in_specs.append(pl.BlockSpec(a.shape, lambda i, nd=a.ndim: (0,) * nd))
    return pl.pallas_call(
        functools.partial(_samlp_body, len(ws)),
        grid=grid,
        in_specs=in_specs,
        out_specs=pl.BlockSpec((rblk // k, cout), lambda i: (i, 0)),
        out_shape=jax.ShapeDtypeStruct((m // k, cout), jnp.float32),
    )(g, nx_rows, *wargs)


def _top3_body(n2, x1_ref, x2_ref, w_ref, idx_ref):
    bi = pl.program_id(0)
    x1 = x1_ref[0]
    x2 = x2_ref[0]
    sn = jnp.sum(x1 * x1, axis=1, keepdims=True)
    xn = jnp.sum(x2 * x2, axis=0, keepdims=True)
    mm = jnp.dot(x1, x2, preferred_element_type=jnp.float32)
    d = sn + xn - 2.0 * mm
    lanes = lax.broadcasted_iota(jnp.int32, (1, n2), 1)
    vals, idxs = [], []
    for _ in range(3):
        mk = jnp.min(d, axis=1, keepdims=True)
        ik = jnp.min(jnp.where(d == mk, lanes, n2), axis=1, keepdims=True)
        d = jnp.where(lanes == ik, jnp.float32(np.inf), d)
        vals.append(mk)
        idxs.append(ik)
    recips = [1.0 / (v + 1e-8) for v in vals]
    norm = recips[0] + recips[1] + recips[2]
    w_ref[0] = jnp.concatenate([r / norm for r in recips], axis=1)
    idx_ref[0] = jnp.concatenate(idxs, axis=1) + bi * n2


def _top3_call(xyz1_rows, xyz2):
    n1 = xyz1_rows.shape[1]
    n2 = xyz2.shape[2]
    n1blk = min(n1, 1024)
    grid = (B, n1 // n1blk)
    return pl.pallas_call(
        functools.partial(_top3_body, n2),
        grid=grid,
        in_specs=[pl.BlockSpec((1, n1blk, 3), lambda b, i: (b, i, 0)),
                  pl.BlockSpec((1, 3, n2), lambda b, i: (b, 0, 0))],
        out_specs=[pl.BlockSpec((1, n1blk, 3), lambda b, i: (b, i, 0)),
                   pl.BlockSpec((1, n1blk, 3), lambda b, i: (b, i, 0))],
        out_shape=[jax.ShapeDtypeStruct((B, n1, 3), jnp.float32),
                   jax.ShapeDtypeStruct((B, n1, 3), jnp.int32)],
    )(xyz1_rows, xyz2)


def _fpmlp_body(nlayer, has_p1, has_head, g_ref, w_ref, *args):
    pos = 0
    if has_p1:
        p1_ref = args[0]
        pos = 1
    wrefs = args[pos:pos + 2 * nlayer + (4 if has_head else 0)]
    out_ref = args[pos + len(wrefs)]
    g = g_ref[...]
    rb3, dd = g.shape
    rb = rb3 // 3
    w = w_ref[...]
    gw = (g * w).reshape(rb, 3, dd)
    interp = gw[:, 0, :] + gw[:, 1, :] + gw[:, 2, :]
    if has_p1:
        h = jnp.concatenate([p1_ref[...], interp], axis=1)
    else:
        h = interp
    for li in range(nlayer):
        wt = wrefs[2 * li][...]
        b = wrefs[2 * li + 1][...]
        h = jnp.maximum(_mm(h, wt) + b, 0.0)
    if has_head:
        hw1, hb1, hw2, hb2 = [wrefs[2 * nlayer + i][...] for i in range(4)]
        h = jnp.maximum(_mm(h, hw1) + hb1, 0.0)
        z = _mm(h, hw2) + hb2
        zm = jnp.max(z, axis=1, keepdims=True)
        sh = z - zm
        h = sh - jnp.log(jnp.sum(jnp.exp(sh), axis=1, keepdims=True))
    out_ref[...] = h


def _fpmlp_call(g, w_rows, p1_rows, ws, head=None):
    m3, dd = g.shape
    rows = m3 // 3
    rblk = min(rows, 2048)
    grid = (rows // rblk,)
    c1 = p1_rows.shape[1] if p1_rows is not None else 0
    wargs = []
    cin = c1 + dd
    for (wt, b) in ws:
        wp = jnp.pad(wt, ((0, 0), (0, cin - wt.shape[1]))) if wt.shape[1] < cin else wt
        wargs += [wp, b.reshape(1, -1)]
        cin = wt.shape[0]
    cout = ws[-1][0].shape[0]
    if head is not None:
        hw1, hb1, hw2, hb2 = head
        wargs += [hw1, hb1.reshape(1, -1), hw2, hb2.reshape(1, -1)]
        cout = hw2.shape[0]
    args = [g, w_rows] + ([p1_rows] if p1_rows is not None else []) + wargs
    in_specs = [pl.BlockSpec((rblk * 3, dd), lambda i: (i, 0)),
                pl.BlockSpec((rblk * 3, 1), lambda i: (i, 0))]
    if p1_rows is not None:
        in_specs.append(pl.BlockSpec((rblk, c1), lambda i: (i, 0)))
    for a in wargs:
        in_specs.append(pl.BlockSpec(a.shape, lambda i, nd=a.ndim: (0,) * nd))
    return pl.pallas_call(
        functools.partial(_fpmlp_body, len(ws), p1_rows is not None,
                          head is not None),
        grid=grid,
        in_specs=in_specs,
        out_specs=pl.BlockSpec((rblk, cout), lambda i: (i, 0)),
        out_shape=jax.ShapeDtypeStruct((rows, cout), jnp.float32),
    )(*args)


def _pad_cols(a, mult=16):
    c = a.shape[1]
    pc = -c % mult
    if pc:
        a = jnp.pad(a, ((0, 0), (0, pc)))
    return a


def _mlp_params(params, name, nl):
    return [(params[name + '_mlp%d_w' % i], params[name + '_mlp%d_b' % i])
            for i in range(nl)]


def _ballq_tmp(dists, r2, s, n, nsample=32):
    r = dists.shape[0]
    gi = jnp.broadcast_to(jnp.arange(n, dtype=jnp.int32), (r, n))
    gi = jnp.where(dists > r2, n, gi)
    gi = jnp.sort(gi, axis=-1)[:, :nsample]
    first = gi[:, :1]
    gi = jnp.where(gi == n, jnp.broadcast_to(first, gi.shape), gi)
    gi = jnp.minimum(gi, n - 1)
    off = ((jnp.arange(r, dtype=jnp.int32) // s) * n)[:, None]
    return gi + off


def _sa_level(l_xyz, l_pts, s, radius, ws):
    n = l_xyz.shape[2]
    c = l_pts.shape[1]
    table = jnp.concatenate(
        [jnp.transpose(l_xyz, (0, 2, 1)).reshape(B * n, 3),
         jnp.transpose(l_pts, (0, 2, 1)).reshape(B * n, c)], axis=1)
    table = _pad_cols(table)
    fps_idx = _fps_call(l_xyz, s).reshape(-1)
    npad = -fps_idx.shape[0] % (8 * NW)
    fps_idx_p = jnp.pad(fps_idx, (0, npad)) if npad else fps_idx
    new_xyz = _sc_gather(table, fps_idx_p)[:B * s, :3].reshape(B, s, 3)
    dists = _sqdist_call(new_xyz, l_xyz)
    gidx = _ballq_call(dists, radius * radius, s, n)
    g = _sc_gather(table, gidx.reshape(-1))
    feats = _samlp_call(g, new_xyz.reshape(B * s, 3), ws)
    new_l_xyz = jnp.transpose(new_xyz, (0, 2, 1))
    new_pts = jnp.transpose(feats.reshape(B, s, -1), (0, 2, 1))
    return new_l_xyz, new_pts


def _fp_level(xyz1, xyz2, pts1, pts2, ws, head=None):
    n1 = xyz1.shape[2]
    n2 = xyz2.shape[2]
    c2 = pts2.shape[1]
    w3, idx3 = _top3_call(jnp.transpose(xyz1, (0, 2, 1)), xyz2)
    table = _pad_cols(jnp.transpose(pts2, (0, 2, 1)).reshape(B * n2, c2))
    g = _sc_gather(table, idx3.reshape(-1))
    p1_rows = None
    if pts1 is not None:
        p1_rows = jnp.transpose(pts1, (0, 2, 1)).reshape(B * n1, -1)
    out = _fpmlp_call(g, w3.reshape(B * n1 * 3, 1), p1_rows, ws, head)
    return out, n1


def kernel(xyz, input_for_alignment_all_structure, params):
    xyz = xyz.astype(jnp.float32)
    n = xyz.shape[2]
    trans, l0_points = _stn_call(xyz, params)
    l0_xyz = xyz[:, :3, :]

    l1_xyz, l1_points = _sa_level(l0_xyz, l0_points, 1024, 0.1,
                                  _mlp_params(params, 'sa1', 3))
    l2_xyz, l2_points = _sa_level(l1_xyz, l1_points, 256, 0.2,
                                  _mlp_params(params, 'sa2', 3))
    l3_xyz, l3_points = _sa_level(l2_xyz, l2_points, 64, 0.4,
                                  _mlp_params(params, 'sa3', 3))
    l4_xyz, l4_points = _sa_level(l3_xyz, l3_points, 16, 0.8,
                                  _mlp_params(params, 'sa4', 3))

    o, n1 = _fp_level(l3_xyz, l4_xyz, l3_points, l4_points,
                      _mlp_params(params, 'fp4', 2))
    fp4_pts = jnp.transpose(o.reshape(B, n1, -1), (0, 2, 1))
    o, n1 = _fp_level(l2_xyz, l3_xyz, l2_points, fp4_pts,
                      _mlp_params(params, 'fp3', 2))
    fp3_pts = jnp.transpose(o.reshape(B, n1, -1), (0, 2, 1))
    o, n1 = _fp_level(l1_xyz, l2_xyz, l1_points, fp3_pts,
                      _mlp_params(params, 'fp2', 2))
    fp2_pts = jnp.transpose(o.reshape(B, n1, -1), (0, 2, 1))
    head = (params['head_conv1_w'], params['head_conv1_b'],
            params['head_conv2_w'], params['head_conv2_b'])
    o, n1 = _fp_level(l0_xyz, l1_xyz, None, fp2_pts,
                      _mlp_params(params, 'fp1', 3), head=head)
    x = o.reshape(B, n, 7)
    return (x, trans)


# speedup vs baseline: 1.2149x; 1.0255x over previous
import functools
import math

import jax
import jax.numpy as jnp
import numpy as np
from jax import lax
from jax.experimental import pallas as pl
from jax.experimental.pallas import tpu as pltpu
from jax.experimental.pallas import tpu_sc as plsc

B = 4
NW = 32
L = 16


def _mm(a, w):
    return lax.dot_general(a, w, (((1,), (1,)), ((), ())),
                           preferred_element_type=jnp.float32)


def _stn_body(x_ref, w1, b1, w2, b2, w3, b3, fw1, fb1, fw2, fb2, fw3, fb3,
              t9_ref, xp_ref):
    x = x_ref[0]
    n = x.shape[1]
    nch = 4
    ch = n // nch
    mx = None
    for ci in range(nch):
        xc = x[:, ci * ch:(ci + 1) * ch]
        h = jnp.maximum(jnp.dot(w1[...], xc, preferred_element_type=jnp.float32) + b1[...], 0.0)
        h = jnp.maximum(jnp.dot(w2[...], h, preferred_element_type=jnp.float32) + b2[...], 0.0)
        h = jnp.maximum(jnp.dot(w3[...], h, preferred_element_type=jnp.float32) + b3[...], 0.0)
        hm = jnp.max(h, axis=1, keepdims=True)
        mx = hm if mx is None else jnp.maximum(mx, hm)
    h = jnp.maximum(jnp.dot(fw1[...], mx, preferred_element_type=jnp.float32) + fb1[...], 0.0)
    h = jnp.maximum(jnp.dot(fw2[...], h, preferred_element_type=jnp.float32) + fb2[...], 0.0)
    t9 = jnp.dot(fw3[...], h, preferred_element_type=jnp.float32) + fb3[...]
    iden = (lax.broadcasted_iota(jnp.int32, (9, 1), 0) % 4 == 0).astype(jnp.float32)
    t9 = t9 + iden
    t9_ref[0] = t9
    tb = t9.astype(jnp.bfloat16).astype(jnp.float32)
    xb = x.astype(jnp.bfloat16).astype(jnp.float32)
    rows = []
    for j in range(3):
        r = (tb[j:j + 1, :] * xb[0:1, :]
             + tb[3 + j:4 + j, :] * xb[1:2, :]
             + tb[6 + j:7 + j, :] * xb[2:3, :])
        rows.append(r)
    xp_ref[0] = jnp.concatenate(rows, axis=0)


def _stn_call(xyz, params):
    n = xyz.shape[2]
    cb = lambda a: a.reshape(-1, 1)
    args = [xyz,
            params['stn_conv1_w'], cb(params['stn_conv1_b']),
            params['stn_conv2_w'], cb(params['stn_conv2_b']),
            params['stn_conv3_w'], cb(params['stn_conv3_b']),
            params['stn_fc1_w'], cb(params['stn_fc1_b']),
            params['stn_fc2_w'], cb(params['stn_fc2_b']),
            params['stn_fc3_w'], cb(params['stn_fc3_b'])]
    in_specs = [pl.BlockSpec((1, 3, n), lambda b: (b, 0, 0))]
    for a in args[1:]:
        in_specs.append(pl.BlockSpec(a.shape, lambda b, nd=a.ndim: (0,) * nd))
    t9, xp = pl.pallas_call(
        _stn_body,
        grid=(B,),
        in_specs=in_specs,
        out_specs=[pl.BlockSpec((1, 9, 1), lambda b: (b, 0, 0)),
                   pl.BlockSpec((1, 3, n), lambda b: (b, 0, 0))],
        out_shape=[jax.ShapeDtypeStruct((B, 9, 1), jnp.float32),
                   jax.ShapeDtypeStruct((B, 3, n), jnp.float32)],
    )(*args)
    return t9.reshape(B, 3, 3), xp


def _fps_body(n, s, xyz_v, idx_ref):
    ln = min(n, 128)
    rn = n // ln
    lin = lax.broadcasted_iota(jnp.int32, (rn, ln), 0) * ln + \
        lax.broadcasted_iota(jnp.int32, (rn, ln), 1)
    xs = [[xyz_v[b, c, :].reshape(rn, ln) for c in range(3)] for b in range(B)]

    def body(i, carry):
        fars, dists = carry
        new_fars = []
        new_dists = []
        for b in range(B):
            f = fars[b]
            idx_ref[b, i] = f + b * n
            fl = lax.rem(f, ln)
            lanes = lax.iota(jnp.int32, ln)
            if rn == 1:
                row = xyz_v[b, :, :]
            else:
                fb = pl.multiple_of(f - fl, ln)
                row = xyz_v[b, :, pl.ds(fb, ln)]
            sel = jnp.where(lanes[None, :] == fl, row, 0.0)
            cx = jnp.sum(sel[0, :])
            cy = jnp.sum(sel[1, :])
            cz = jnp.sum(sel[2, :])
            d = ((xs[b][0] - cx) ** 2 + (xs[b][1] - cy) ** 2
                 + (xs[b][2] - cz) ** 2)
            d = jnp.minimum(dists[b], d)
            m = jnp.max(d)
            far = jnp.min(jnp.where(d == m, lin, n))
            new_fars.append(far)
            new_dists.append(d)
        return tuple(new_fars), tuple(new_dists)

    fars0 = tuple(jnp.int32(0) for _ in range(B))
    dists0 = tuple(jnp.full((rn, ln), 1e10, jnp.float32) for _ in range(B))
    lax.fori_loop(0, s, body, (fars0, dists0))


def _fps_call(l_xyz, s):
    n = l_xyz.shape[2]
    return pl.pallas_call(
        functools.partial(_fps_body, n, s),
        out_specs=pl.BlockSpec(memory_space=pltpu.SMEM),
        out_shape=jax.ShapeDtypeStruct((B, s), jnp.int32),
    )(l_xyz)


def _sqdist_body(nx_ref, xt_ref, out_ref):
    nx = nx_ref[0]
    xt = xt_ref[0]
    sn = jnp.sum(nx * nx, axis=1, keepdims=True)
    xn = jnp.sum(xt * xt, axis=0, keepdims=True)
    mm = jnp.dot(nx, xt, preferred_element_type=jnp.float32)
    out_ref[...] = sn + xn - 2.0 * mm


def _sqdist_call(new_xyz, l_xyz):
    s = new_xyz.shape[1]
    n = l_xyz.shape[2]
    sblk = min(s, 256)
    grid = (B, s // sblk)
    return pl.pallas_call(
        _sqdist_body,
        grid=grid,
        in_specs=[pl.BlockSpec((1, sblk, 3), lambda b, i: (b, i, 0)),
                  pl.BlockSpec((1, 3, n), lambda b, i: (b, 0, 0))],
        out_specs=pl.BlockSpec((sblk, n), lambda b, i: (b * (s // sblk) + i, 0)),
        out_shape=jax.ShapeDtypeStruct((B * s, n), jnp.float32),
    )(new_xyz, l_xyz)


def _ballq_call(dists, r2, s, n, nsample=32):
    r = dists.shape[0]
    r_w = r // NW
    nchunk = n // L
    mesh = plsc.VectorSubcoreMesh(core_axis_name="c", subcore_axis_name="s")
    log2s = int(math.log2(s))
    _GRP = min(8, r_w)

    def body(d_hbm, out_hbm, d_v, o_v):
        wid = lax.axis_index("s") * 2 + lax.axis_index("c")

        def group_step(gj, _):
            grow = wid * r_w + gj * _GRP
            pltpu.sync_copy(d_hbm.at[pl.ds(grow, _GRP)], d_v)
            for bj in range(_GRP):
                row_body(bj, grow + bj)
            return 0

        def row_body(bj, row):
            off = (row >> log2s) * n
            lanes = lax.iota(jnp.int32, L)

            big = jnp.int32(2 ** 30)

            def chunk_step(ck, carry):
                cnt_v, fv = carry
                d = d_v[bj, pl.ds(ck * L, L)]
                m = d <= r2
                keys = jnp.where(m, lanes, 2 * L)
                pos = lanes + (ck * L + off)
                _, sv = plsc.sort_key_val(keys, pos)
                sidx = jnp.minimum(cnt_v, nsample) + lanes
                plsc.store_scatter(o_v, [sidx], sv)
                fv = jnp.minimum(fv, jnp.where(m, pos, big))
                return cnt_v + plsc.all_reduce_population_count(m), fv

            cnt_v, fv = lax.fori_loop(
                0, nchunk, chunk_step,
                (jnp.zeros((L,), jnp.int32), jnp.full((L,), 2 ** 30, jnp.int32)),
                unroll=4)
            for k in (1, 2, 4, 8):
                fv = jnp.minimum(
                    fv, fv.at[lanes ^ k].get(mode="promise_in_bounds"))
            first = jnp.where(cnt_v > 0, fv, n - 1 + off)
            for half in range(nsample // L):
                ids = lanes + half * L
                cur = o_v[pl.ds(half * L, L)]
                o_v[pl.ds(half * L, L)] = jnp.where(ids < cnt_v, cur, first)
            pltpu.sync_copy(o_v.at[pl.ds(0, nsample)], out_hbm.at[row])

        lax.fori_loop(0, r_w // _GRP, group_step, 0)

    f = pl.kernel(
        body,
        out_type=jax.ShapeDtypeStruct((r, nsample), jnp.int32),
        mesh=mesh,
        compiler_params=pltpu.CompilerParams(use_tc_tiling_on_sc=False,
                                             needs_layout_passes=False),
        scratch_types=[pltpu.VMEM((_GRP, n), jnp.float32),
                       pltpu.VMEM((nsample + L,), jnp.int32)],
    )
    return f(dists)


def _sc_gather(table, idx):
    m = idx.shape[0]
    d = table.shape[1]
    m_w = m // NW
    chunk = m_w
    while chunk * d * 4 > 320 * 1024:
        chunk //= 2
    nchunks = m_w // chunk
    mesh = plsc.VectorSubcoreMesh(core_axis_name="c", subcore_axis_name="s")

    def body(table_hbm, idx_hbm, out_hbm, idx_v, rows_v, sem):
        wid = lax.axis_index("s") * 2 + lax.axis_index("c")
        base = wid * m_w

        def step(ci, _):
            off = base + ci * chunk
            pltpu.sync_copy(idx_hbm.at[pl.ds(off, chunk)], idx_v)
            pltpu.async_copy(table_hbm.at[idx_v], rows_v, sem).wait()
            pltpu.sync_copy(rows_v, out_hbm.at[pl.ds(off, chunk)])
            return 0

        lax.fori_loop(0, nchunks, step, 0)

    f = pl.kernel(
        body,
        out_type=jax.ShapeDtypeStruct((m, d), jnp.float32),
        mesh=mesh,
        compiler_params=pltpu.CompilerParams(use_tc_tiling_on_sc=False),
        scratch_types=[pltpu.VMEM((chunk,), jnp.int32),
                       pltpu.VMEM((chunk, d), jnp.float32),
                       pltpu.SemaphoreType.DMA],
    )
    return f(table, idx)


def _samlp_body(nlayer, g_ref, nx_ref, *args):
    wrefs = args[:2 * nlayer]
    out_ref = args[2 * nlayer]
    g = g_ref[...]
    rblk, d = g.shape
    k = 32
    g3 = g.reshape(rblk // k, k, d)
    nx = nx_ref[...]
    gx = g3[:, :, :3] - nx[:, None, :]
    h = jnp.concatenate([gx, g3[:, :, 3:]], axis=2)
    for li in range(nlayer):
        w = wrefs[2 * li][...]
        b = wrefs[2 * li + 1][...]
        h = _mm(h.reshape(h.shape[0] * k, h.shape[2]), w)
        h = jnp.maximum(h + b, 0.0).reshape(rblk // k, k, w.shape[0])
    out_ref[...] = jnp.max(h, axis=1)


def _samlp_call(g, nx_rows, ws):
    m, d = g.shape
    k = 32
    cout = ws[-1][0].shape[0]
    rblk = min(m, 8192)
    grid = (m // rblk,)
    wargs = []
    for (w, b) in ws:
        wp = jnp.pad(w, ((0, 0), (0, d - w.shape[1]))) if w.shape[1] < d else w
        wargs += [wp, b.reshape(1, -1)]
        d = w.shape[0]
    in_specs = [pl.BlockSpec((rblk, g.shape[1]), lambda i: (i, 0)),
                pl.BlockSpec((rblk // k, 3), lambda i: (i, 0))]
    for a in wargs:
        in_specs.append(pl.BlockSpec(a.shape, lambda i, nd=a.ndim: (0,) * nd))
    return pl.pallas_call(
        functools.partial(_samlp_body, len(ws)),
        grid=grid,
        in_specs=in_specs,
        out_specs=pl.BlockSpec((rblk // k, cout), lambda i: (i, 0)),
        out_shape=jax.ShapeDtypeStruct((m // k, cout), jnp.float32),
    )(g, nx_rows, *wargs)


def _top3_body(n2, x1_ref, x2_ref, w_ref, idx_ref):
    bi = pl.program_id(0)
    x1 = x1_ref[0]
    x2 = x2_ref[0]
    sn = jnp.sum(x1 * x1, axis=1, keepdims=True)
    xn = jnp.sum(x2 * x2, axis=0, keepdims=True)
    mm = jnp.dot(x1, x2, preferred_element_type=jnp.float32)
    d = sn + xn - 2.0 * mm
    lanes = lax.broadcasted_iota(jnp.int32, (1, n2), 1)
    vals, idxs = [], []
    for _ in range(3):
        mk = jnp.min(d, axis=1, keepdims=True)
        ik = jnp.min(jnp.where(d == mk, lanes, n2), axis=1, keepdims=True)
        d = jnp.where(lanes == ik, jnp.float32(np.inf), d)
        vals.append(mk)
        idxs.append(ik)
    recips = [1.0 / (v + 1e-8) for v in vals]
    norm = recips[0] + recips[1] + recips[2]
    w_ref[0] = jnp.concatenate([r / norm for r in recips], axis=1)
    idx_ref[0] = jnp.concatenate(idxs, axis=1) + bi * n2


def _top3_call(xyz1_rows, xyz2):
    n1 = xyz1_rows.shape[1]
    n2 = xyz2.shape[2]
    n1blk = min(n1, 1024)
    grid = (B, n1 // n1blk)
    return pl.pallas_call(
        functools.partial(_top3_body, n2),
        grid=grid,
        in_specs=[pl.BlockSpec((1, n1blk, 3), lambda b, i: (b, i, 0)),
                  pl.BlockSpec((1, 3, n2), lambda b, i: (b, 0, 0))],
        out_specs=[pl.BlockSpec((1, n1blk, 3), lambda b, i: (b, i, 0)),
                   pl.BlockSpec((1, n1blk, 3), lambda b, i: (b, i, 0))],
        out_shape=[jax.ShapeDtypeStruct((B, n1, 3), jnp.float32),
                   jax.ShapeDtypeStruct((B, n1, 3), jnp.int32)],
    )(xyz1_rows, xyz2)


def _fpmlp_body(nlayer, has_p1, has_head, g_ref, w_ref, *args):
    pos = 0
    if has_p1:
        p1_ref = args[0]
        pos = 1
    wrefs = args[pos:pos + 2 * nlayer + (4 if has_head else 0)]
    out_ref = args[pos + len(wrefs)]
    g = g_ref[...]
    rb3, dd = g.shape
    rb = rb3 // 3
    w = w_ref[...]
    gw = (g * w).reshape(rb, 3, dd)
    interp = gw[:, 0, :] + gw[:, 1, :] + gw[:, 2, :]
    if has_p1:
        h = jnp.concatenate([p1_ref[...], interp], axis=1)
    else:
        h = interp
    for li in range(nlayer):
        wt = wrefs[2 * li][...]
        b = wrefs[2 * li + 1][...]
        h = jnp.maximum(_mm(h, wt) + b, 0.0)
    if has_head:
        hw1, hb1, hw2, hb2 = [wrefs[2 * nlayer + i][...] for i in range(4)]
        h = jnp.maximum(_mm(h, hw1) + hb1, 0.0)
        z = _mm(h, hw2) + hb2
        zm = jnp.max(z, axis=1, keepdims=True)
        sh = z - zm
        h = sh - jnp.log(jnp.sum(jnp.exp(sh), axis=1, keepdims=True))
    out_ref[...] = h


def _fpmlp_call(g, w_rows, p1_rows, ws, head=None):
    m3, dd = g.shape
    rows = m3 // 3
    rblk = min(rows, 2048)
    grid = (rows // rblk,)
    c1 = p1_rows.shape[1] if p1_rows is not None else 0
    wargs = []
    cin = c1 + dd
    for (wt, b) in ws:
        wp = jnp.pad(wt, ((0, 0), (0, cin - wt.shape[1]))) if wt.shape[1] < cin else wt
        wargs += [wp, b.reshape(1, -1)]
        cin = wt.shape[0]
    cout = ws[-1][0].shape[0]
    if head is not None:
        hw1, hb1, hw2, hb2 = head
        wargs += [hw1, hb1.reshape(1, -1), hw2, hb2.reshape(1, -1)]
        cout = hw2.shape[0]
    args = [g, w_rows] + ([p1_rows] if p1_rows is not None else []) + wargs
    in_specs = [pl.BlockSpec((rblk * 3, dd), lambda i: (i, 0)),
                pl.BlockSpec((rblk * 3, 1), lambda i: (i, 0))]
    if p1_rows is not None:
        in_specs.append(pl.BlockSpec((rblk, c1), lambda i: (i, 0)))
    for a in wargs:
        in_specs.append(pl.BlockSpec(a.shape, lambda i, nd=a.ndim: (0,) * nd))
    return pl.pallas_call(
        functools.partial(_fpmlp_body, len(ws), p1_rows is not None,
                          head is not None),
        grid=grid,
        in_specs=in_specs,
        out_specs=pl.BlockSpec((rblk, cout), lambda i: (i, 0)),
        out_shape=jax.ShapeDtypeStruct((rows, cout), jnp.float32),
    )(*args)


def _pad_cols(a, mult=16):
    c = a.shape[1]
    pc = -c % mult
    if pc:
        a = jnp.pad(a, ((0, 0), (0, pc)))
    return a


def _mlp_params(params, name, nl):
    return [(params[name + '_mlp%d_w' % i], params[name + '_mlp%d_b' % i])
            for i in range(nl)]


def _ballq_tmp(dists, r2, s, n, nsample=32):
    r = dists.shape[0]
    gi = jnp.broadcast_to(jnp.arange(n, dtype=jnp.int32), (r, n))
    gi = jnp.where(dists > r2, n, gi)
    gi = jnp.sort(gi, axis=-1)[:, :nsample]
    first = gi[:, :1]
    gi = jnp.where(gi == n, jnp.broadcast_to(first, gi.shape), gi)
    gi = jnp.minimum(gi, n - 1)
    off = ((jnp.arange(r, dtype=jnp.int32) // s) * n)[:, None]
    return gi + off


def _sa_level(l_xyz, l_pts, s, radius, ws):
    n = l_xyz.shape[2]
    c = l_pts.shape[1]
    table = jnp.concatenate(
        [jnp.transpose(l_xyz, (0, 2, 1)).reshape(B * n, 3),
         jnp.transpose(l_pts, (0, 2, 1)).reshape(B * n, c)], axis=1)
    table = _pad_cols(table)
    fps_idx = _fps_call(l_xyz, s).reshape(-1)
    npad = -fps_idx.shape[0] % (8 * NW)
    fps_idx_p = jnp.pad(fps_idx, (0, npad)) if npad else fps_idx
    new_xyz = _sc_gather(table, fps_idx_p)[:B * s, :3].reshape(B, s, 3)
    dists = _sqdist_call(new_xyz, l_xyz)
    gidx = _ballq_call(dists, radius * radius, s, n)
    g = _sc_gather(table, gidx.reshape(-1))
    feats = _samlp_call(g, new_xyz.reshape(B * s, 3), ws)
    new_l_xyz = jnp.transpose(new_xyz, (0, 2, 1))
    new_pts = jnp.transpose(feats.reshape(B, s, -1), (0, 2, 1))
    return new_l_xyz, new_pts


def _fp_level(xyz1, xyz2, pts1, pts2, ws, head=None):
    n1 = xyz1.shape[2]
    n2 = xyz2.shape[2]
    c2 = pts2.shape[1]
    w3, idx3 = _top3_call(jnp.transpose(xyz1, (0, 2, 1)), xyz2)
    table = _pad_cols(jnp.transpose(pts2, (0, 2, 1)).reshape(B * n2, c2))
    g = _sc_gather(table, idx3.reshape(-1))
    p1_rows = None
    if pts1 is not None:
        p1_rows = jnp.transpose(pts1, (0, 2, 1)).reshape(B * n1, -1)
    out = _fpmlp_call(g, w3.reshape(B * n1 * 3, 1), p1_rows, ws, head)
    return out, n1


def kernel(xyz, input_for_alignment_all_structure, params):
    xyz = xyz.astype(jnp.float32)
    n = xyz.shape[2]
    trans, l0_points = _stn_call(xyz, params)
    l0_xyz = xyz[:, :3, :]

    l1_xyz, l1_points = _sa_level(l0_xyz, l0_points, 1024, 0.1,
                                  _mlp_params(params, 'sa1', 3))
    l2_xyz, l2_points = _sa_level(l1_xyz, l1_points, 256, 0.2,
                                  _mlp_params(params, 'sa2', 3))
    l3_xyz, l3_points = _sa_level(l2_xyz, l2_points, 64, 0.4,
                                  _mlp_params(params, 'sa3', 3))
    l4_xyz, l4_points = _sa_level(l3_xyz, l3_points, 16, 0.8,
                                  _mlp_params(params, 'sa4', 3))

    o, n1 = _fp_level(l3_xyz, l4_xyz, l3_points, l4_points,
                      _mlp_params(params, 'fp4', 2))
    fp4_pts = jnp.transpose(o.reshape(B, n1, -1), (0, 2, 1))
    o, n1 = _fp_level(l2_xyz, l3_xyz, l2_points, fp4_pts,
                      _mlp_params(params, 'fp3', 2))
    fp3_pts = jnp.transpose(o.reshape(B, n1, -1), (0, 2, 1))
    o, n1 = _fp_level(l1_xyz, l2_xyz, l1_points, fp3_pts,
                      _mlp_params(params, 'fp2', 2))
    fp2_pts = jnp.transpose(o.reshape(B, n1, -1), (0, 2, 1))
    head = (params['head_conv1_w'], params['head_conv1_b'],
            params['head_conv2_w'], params['head_conv2_b'])
    o, n1 = _fp_level(l0_xyz, l1_xyz, None, fp2_pts,
                      _mlp_params(params, 'fp1', 3), head=head)
    x = o.reshape(B, n, 7)
    return (x, trans)


# speedup vs baseline: 1.2187x; 1.0031x over previous
import functools
import math

import jax
import jax.numpy as jnp
import numpy as np
from jax import lax
from jax.experimental import pallas as pl
from jax.experimental.pallas import tpu as pltpu
from jax.experimental.pallas import tpu_sc as plsc

B = 4
NW = 32
L = 16


def _mm(a, w):
    return lax.dot_general(a, w, (((1,), (1,)), ((), ())),
                           preferred_element_type=jnp.float32)


def _stn_body(x_ref, w1, b1, w2, b2, w3, b3, fw1, fb1, fw2, fb2, fw3, fb3,
              t9_ref, xp_ref):
    x = x_ref[0]
    n = x.shape[1]
    nch = 4
    ch = n // nch
    mx = None
    for ci in range(nch):
        xc = x[:, ci * ch:(ci + 1) * ch]
        h = jnp.maximum(jnp.dot(w1[...], xc, preferred_element_type=jnp.float32) + b1[...], 0.0)
        h = jnp.maximum(jnp.dot(w2[...], h, preferred_element_type=jnp.float32) + b2[...], 0.0)
        h = jnp.maximum(jnp.dot(w3[...], h, preferred_element_type=jnp.float32) + b3[...], 0.0)
        hm = jnp.max(h, axis=1, keepdims=True)
        mx = hm if mx is None else jnp.maximum(mx, hm)
    h = jnp.maximum(jnp.dot(fw1[...], mx, preferred_element_type=jnp.float32) + fb1[...], 0.0)
    h = jnp.maximum(jnp.dot(fw2[...], h, preferred_element_type=jnp.float32) + fb2[...], 0.0)
    t9 = jnp.dot(fw3[...], h, preferred_element_type=jnp.float32) + fb3[...]
    iden = (lax.broadcasted_iota(jnp.int32, (9, 1), 0) % 4 == 0).astype(jnp.float32)
    t9 = t9 + iden
    t9_ref[0] = t9
    tb = t9.astype(jnp.bfloat16).astype(jnp.float32)
    xb = x.astype(jnp.bfloat16).astype(jnp.float32)
    rows = []
    for j in range(3):
        r = (tb[j:j + 1, :] * xb[0:1, :]
             + tb[3 + j:4 + j, :] * xb[1:2, :]
             + tb[6 + j:7 + j, :] * xb[2:3, :])
        rows.append(r)
    xp_ref[0] = jnp.concatenate(rows, axis=0)


def _stn_call(xyz, params):
    n = xyz.shape[2]
    cb = lambda a: a.reshape(-1, 1)
    args = [xyz,
            params['stn_conv1_w'], cb(params['stn_conv1_b']),
            params['stn_conv2_w'], cb(params['stn_conv2_b']),
            params['stn_conv3_w'], cb(params['stn_conv3_b']),
            params['stn_fc1_w'], cb(params['stn_fc1_b']),
            params['stn_fc2_w'], cb(params['stn_fc2_b']),
            params['stn_fc3_w'], cb(params['stn_fc3_b'])]
    in_specs = [pl.BlockSpec((1, 3, n), lambda b: (b, 0, 0))]
    for a in args[1:]:
        in_specs.append(pl.BlockSpec(a.shape, lambda b, nd=a.ndim: (0,) * nd))
    t9, xp = pl.pallas_call(
        _stn_body,
        grid=(B,),
        in_specs=in_specs,
        out_specs=[pl.BlockSpec((1, 9, 1), lambda b: (b, 0, 0)),
                   pl.BlockSpec((1, 3, n), lambda b: (b, 0, 0))],
        out_shape=[jax.ShapeDtypeStruct((B, 9, 1), jnp.float32),
                   jax.ShapeDtypeStruct((B, 3, n), jnp.float32)],
    )(*args)
    return t9.reshape(B, 3, 3), xp


def _fps_body(n, s, xyz_v, idx_ref):
    ln = min(n, 128)
    rn = n // ln
    lin = lax.broadcasted_iota(jnp.int32, (rn, ln), 0) * ln + \
        lax.broadcasted_iota(jnp.int32, (rn, ln), 1)
    xs = [[xyz_v[b, c, :].reshape(rn, ln) for c in range(3)] for b in range(B)]

    def body(i, carry):
        fars, dists = carry
        new_fars = []
        new_dists = []
        for b in range(B):
            f = fars[b]
            idx_ref[b, i] = f + b * n
            fl = lax.rem(f, ln)
            lanes = lax.iota(jnp.int32, ln)
            if rn == 1:
                row = xyz_v[b, :, :]
            else:
                fb = pl.multiple_of(f - fl, ln)
                row = xyz_v[b, :, pl.ds(fb, ln)]
            sel = jnp.where(lanes[None, :] == fl, row, 0.0)
            cx = jnp.sum(sel[0, :])
            cy = jnp.sum(sel[1, :])
            cz = jnp.sum(sel[2, :])
            d = ((xs[b][0] - cx) ** 2 + (xs[b][1] - cy) ** 2
                 + (xs[b][2] - cz) ** 2)
            d = jnp.minimum(dists[b], d)
            m = jnp.max(d)
            far = jnp.min(jnp.where(d == m, lin, n))
            new_fars.append(far)
            new_dists.append(d)
        return tuple(new_fars), tuple(new_dists)

    fars0 = tuple(jnp.int32(0) for _ in range(B))
    dists0 = tuple(jnp.full((rn, ln), 1e10, jnp.float32) for _ in range(B))
    lax.fori_loop(0, s, body, (fars0, dists0))


def _fps_call(l_xyz, s):
    n = l_xyz.shape[2]
    return pl.pallas_call(
        functools.partial(_fps_body, n, s),
        out_specs=pl.BlockSpec(memory_space=pltpu.SMEM),
        out_shape=jax.ShapeDtypeStruct((B, s), jnp.int32),
    )(l_xyz)


def _sqdist_body(nx_ref, xt_ref, out_ref):
    nx = nx_ref[0]
    xt = xt_ref[0]
    sn = jnp.sum(nx * nx, axis=1, keepdims=True)
    xn = jnp.sum(xt * xt, axis=0, keepdims=True)
    mm = jnp.dot(nx, xt, preferred_element_type=jnp.float32)
    out_ref[...] = sn + xn - 2.0 * mm


def _sqdist_call(new_xyz, l_xyz):
    s = new_xyz.shape[1]
    n = l_xyz.shape[2]
    sblk = min(s, 256)
    grid = (B, s // sblk)
    return pl.pallas_call(
        _sqdist_body,
        grid=grid,
        in_specs=[pl.BlockSpec((1, sblk, 3), lambda b, i: (b, i, 0)),
                  pl.BlockSpec((1, 3, n), lambda b, i: (b, 0, 0))],
        out_specs=pl.BlockSpec((sblk, n), lambda b, i: (b * (s // sblk) + i, 0)),
        out_shape=jax.ShapeDtypeStruct((B * s, n), jnp.float32),
    )(new_xyz, l_xyz)


def _ballq_call(dists, r2, s, n, nsample=32):
    r = dists.shape[0]
    r_w = r // NW
    nchunk = n // L
    mesh = plsc.VectorSubcoreMesh(core_axis_name="c", subcore_axis_name="s")
    log2s = int(math.log2(s))
    _GRP = min(8, r_w)

    def body(d_hbm, out_hbm, d_v, o_v):
        wid = lax.axis_index("s") * 2 + lax.axis_index("c")

        def group_step(gj, _):
            grow = wid * r_w + gj * _GRP
            pltpu.sync_copy(d_hbm.at[pl.ds(grow, _GRP)], d_v)
            for bj in range(_GRP):
                row_body(bj, grow + bj)
            return 0

        def row_body(bj, row):
            off = (row >> log2s) * n
            lanes = lax.iota(jnp.int32, L)

            big = jnp.int32(2 ** 30)

            def chunk_step(ck, carry):
                cnt_v, fv = carry
                d = d_v[bj, pl.ds(ck * L, L)]
                m = d <= r2
                keys = jnp.where(m, lanes, 2 * L)
                pos = lanes + (ck * L + off)
                _, sv = plsc.sort_key_val(keys, pos)
                sidx = jnp.minimum(cnt_v, nsample) + lanes
                plsc.store_scatter(o_v, [sidx], sv)
                fv = jnp.minimum(fv, jnp.where(m, pos, big))
                return cnt_v + plsc.all_reduce_population_count(m), fv

            cnt_v, fv = lax.fori_loop(
                0, nchunk, chunk_step,
                (jnp.zeros((L,), jnp.int32), jnp.full((L,), 2 ** 30, jnp.int32)),
                unroll=4)
            for k in (1, 2, 4, 8):
                fv = jnp.minimum(
                    fv, fv.at[lanes ^ k].get(mode="promise_in_bounds"))
            first = jnp.where(cnt_v > 0, fv, n - 1 + off)
            for half in range(nsample // L):
                ids = lanes + half * L
                cur = o_v[pl.ds(half * L, L)]
                o_v[pl.ds(half * L, L)] = jnp.where(ids < cnt_v, cur, first)
            pltpu.sync_copy(o_v.at[pl.ds(0, nsample)], out_hbm.at[row])

        lax.fori_loop(0, r_w // _GRP, group_step, 0)

    f = pl.kernel(
        body,
        out_type=jax.ShapeDtypeStruct((r, nsample), jnp.int32),
        mesh=mesh,
        compiler_params=pltpu.CompilerParams(use_tc_tiling_on_sc=False,
                                             needs_layout_passes=False),
        scratch_types=[pltpu.VMEM((_GRP, n), jnp.float32),
                       pltpu.VMEM((nsample + L,), jnp.int32)],
    )
    return f(dists)


def _sc_gather(table, idx):
    m = idx.shape[0]
    d = table.shape[1]
    m_w = m // NW
    chunk = m_w
    while chunk * d * 4 > 320 * 1024:
        chunk //= 2
    nchunks = m_w // chunk
    mesh = plsc.VectorSubcoreMesh(core_axis_name="c", subcore_axis_name="s")

    def body(table_hbm, idx_hbm, out_hbm, idx_v, rows_v, sem):
        wid = lax.axis_index("s") * 2 + lax.axis_index("c")
        base = wid * m_w

        def step(ci, _):
            off = base + ci * chunk
            pltpu.sync_copy(idx_hbm.at[pl.ds(off, chunk)], idx_v)
            pltpu.async_copy(table_hbm.at[idx_v], rows_v, sem).wait()
            pltpu.sync_copy(rows_v, out_hbm.at[pl.ds(off, chunk)])
            return 0

        lax.fori_loop(0, nchunks, step, 0)

    f = pl.kernel(
        body,
        out_type=jax.ShapeDtypeStruct((m, d), jnp.float32),
        mesh=mesh,
        compiler_params=pltpu.CompilerParams(use_tc_tiling_on_sc=False),
        scratch_types=[pltpu.VMEM((chunk,), jnp.int32),
                       pltpu.VMEM((chunk, d), jnp.float32),
                       pltpu.SemaphoreType.DMA],
    )
    return f(table, idx)


def _samlp_body(nlayer, g_ref, nx_ref, *args):
    wrefs = args[:2 * nlayer]
    out_ref = args[2 * nlayer]
    g = g_ref[...]
    rblk, d = g.shape
    k = 32
    g3 = g.reshape(rblk // k, k, d)
    nx = nx_ref[...]
    gx = g3[:, :, :3] - nx[:, None, :]
    h = jnp.concatenate([gx, g3[:, :, 3:]], axis=2)
    for li in range(nlayer):
        w = wrefs[2 * li][...]
        b = wrefs[2 * li + 1][...]
        h = _mm(h.reshape(h.shape[0] * k, h.shape[2]), w)
        h = jnp.maximum(h + b, 0.0).reshape(rblk // k, k, w.shape[0])
    out_ref[...] = jnp.max(h, axis=1)


def _samlp_call(g, nx_rows, ws):
    m, d = g.shape
    k = 32
    cout = ws[-1][0].shape[0]
    rblk = min(m, 8192)
    grid = (m // rblk,)
    wargs = []
    for (w, b) in ws:
        wp = jnp.pad(w, ((0, 0), (0, d - w.shape[1]))) if w.shape[1] < d else w
        wargs += [wp, b.reshape(1, -1)]
        d = w.shape[0]
    in_specs = [pl.BlockSpec((rblk, g.shape[1]), lambda i: (i, 0)),
                pl.BlockSpec((rblk // k, 3), lambda i: (i, 0))]
    for a in wargs:
        in_specs.append(pl.BlockSpec(a.shape, lambda i, nd=a.ndim: (0,) * nd))
    return pl.pallas_call(
        functools.partial(_samlp_body, len(ws)),
        grid=grid,
        in_specs=in_specs,
        out_specs=pl.BlockSpec((rblk // k, cout), lambda i: (i, 0)),
        out_shape=jax.ShapeDtypeStruct((m // k, cout), jnp.float32),
    )(g, nx_rows, *wargs)


def _top3_body(n2, x1_ref, x2_ref, w_ref, idx_ref):
    bi = pl.program_id(0)
    x1 = x1_ref[0]
    x2 = x2_ref[0]
    sn = jnp.sum(x1 * x1, axis=1, keepdims=True)
    xn = jnp.sum(x2 * x2, axis=0, keepdims=True)
    mm = jnp.dot(x1, x2, preferred_element_type=jnp.float32)
    d = sn + xn - 2.0 * mm
    lanes = lax.broadcasted_iota(jnp.int32, (1, n2), 1)
    vals, idxs = [], []
    for _ in range(3):
        mk = jnp.min(d, axis=1, keepdims=True)
        ik = jnp.min(jnp.where(d == mk, lanes, n2), axis=1, keepdims=True)
        d = jnp.where(lanes == ik, jnp.float32(np.inf), d)
        vals.append(mk)
        idxs.append(ik)
    recips = [1.0 / (v + 1e-8) for v in vals]
    norm = recips[0] + recips[1] + recips[2]
    w_ref[0] = jnp.concatenate([r / norm for r in recips], axis=1)
    idx_ref[0] = jnp.concatenate(idxs, axis=1) + bi * n2


def _top3_call(xyz1_rows, xyz2):
    n1 = xyz1_rows.shape[1]
    n2 = xyz2.shape[2]
    n1blk = min(n1, 1024)
    grid = (B, n1 // n1blk)
    return pl.pallas_call(
        functools.partial(_top3_body, n2),
        grid=grid,
        in_specs=[pl.BlockSpec((1, n1blk, 3), lambda b, i: (b, i, 0)),
                  pl.BlockSpec((1, 3, n2), lambda b, i: (b, 0, 0))],
        out_specs=[pl.BlockSpec((1, n1blk, 3), lambda b, i: (b, i, 0)),
                   pl.BlockSpec((1, n1blk, 3), lambda b, i: (b, i, 0))],
        out_shape=[jax.ShapeDtypeStruct((B, n1, 3), jnp.float32),
                   jax.ShapeDtypeStruct((B, n1, 3), jnp.int32)],
    )(xyz1_rows, xyz2)


def _fpmlp_body(nlayer, has_p1, has_head, g_ref, w_ref, *args):
    pos = 0
    if has_p1:
        p1_ref = args[0]
        pos = 1
    wrefs = args[pos:pos + 2 * nlayer + (4 if has_head else 0)]
    out_ref = args[pos + len(wrefs)]
    g = g_ref[...]
    rb3, dd = g.shape
    rb = rb3 // 3
    w = w_ref[...]
    gw = (g * w).reshape(rb, 3, dd)
    interp = gw[:, 0, :] + gw[:, 1, :] + gw[:, 2, :]
    if has_p1:
        h = jnp.concatenate([p1_ref[...], interp], axis=1)
    else:
        h = interp
    for li in range(nlayer):
        wt = wrefs[2 * li][...]
        b = wrefs[2 * li + 1][...]
        h = jnp.maximum(_mm(h, wt) + b, 0.0)
    if has_head:
        hw1, hb1, hw2, hb2 = [wrefs[2 * nlayer + i][...] for i in range(4)]
        h = jnp.maximum(_mm(h, hw1) + hb1, 0.0)
        z = _mm(h, hw2) + hb2
        zm = jnp.max(z, axis=1, keepdims=True)
        sh = z - zm
        h = sh - jnp.log(jnp.sum(jnp.exp(sh), axis=1, keepdims=True))
    out_ref[...] = h


def _fpmlp_call(g, w_rows, p1_rows, ws, head=None):
    m3, dd = g.shape
    rows = m3 // 3
    rblk = min(rows, 2048)
    grid = (rows // rblk,)
    c1 = p1_rows.shape[1] if p1_rows is not None else 0
    wargs = []
    cin = c1 + dd
    for (wt, b) in ws:
        wp = jnp.pad(wt, ((0, 0), (0, cin - wt.shape[1]))) if wt.shape[1] < cin else wt
        wargs += [wp, b.reshape(1, -1)]
        cin = wt.shape[0]
    cout = ws[-1][0].shape[0]
    if head is not None:
        hw1, hb1, hw2, hb2 = head
        wargs += [hw1, hb1.reshape(1, -1), hw2, hb2.reshape(1, -1)]
        cout = hw2.shape[0]
    args = [g, w_rows] + ([p1_rows] if p1_rows is not None else []) + wargs
    in_specs = [pl.BlockSpec((rblk * 3, dd), lambda i: (i, 0)),
                pl.BlockSpec((rblk * 3, 1), lambda i: (i, 0))]
    if p1_rows is not None:
        in_specs.append(pl.BlockSpec((rblk, c1), lambda i: (i, 0)))
    for a in wargs:
        in_specs.append(pl.BlockSpec(a.shape, lambda i, nd=a.ndim: (0,) * nd))
    return pl.pallas_call(
        functools.partial(_fpmlp_body, len(ws), p1_rows is not None,
                          head is not None),
        grid=grid,
        in_specs=in_specs,
        out_specs=pl.BlockSpec((rblk, cout), lambda i: (i, 0)),
        out_shape=jax.ShapeDtypeStruct((rows, cout), jnp.float32),
    )(*args)


def _pad_cols(a, mult=16):
    c = a.shape[1]
    pc = -c % mult
    if pc:
        a = jnp.pad(a, ((0, 0), (0, pc)))
    return a


def _mlp_params(params, name, nl):
    return [(params[name + '_mlp%d_w' % i], params[name + '_mlp%d_b' % i])
            for i in range(nl)]


def _sa_level(l_xyz, l_pts, s, radius, ws):
    n = l_xyz.shape[2]
    c = l_pts.shape[1]
    table = jnp.concatenate(
        [jnp.transpose(l_xyz, (0, 2, 1)).reshape(B * n, 3),
         jnp.transpose(l_pts, (0, 2, 1)).reshape(B * n, c)], axis=1)
    table = _pad_cols(table)
    fps_idx = _fps_call(l_xyz, s).reshape(-1)
    npad = -fps_idx.shape[0] % (8 * NW)
    fps_idx_p = jnp.pad(fps_idx, (0, npad)) if npad else fps_idx
    new_xyz = _sc_gather(table, fps_idx_p)[:B * s, :3].reshape(B, s, 3)
    dists = _sqdist_call(new_xyz, l_xyz)
    gidx = _ballq_call(dists, radius * radius, s, n)
    g = _sc_gather(table, gidx.reshape(-1))
    feats = _samlp_call(g, new_xyz.reshape(B * s, 3), ws)
    new_l_xyz = jnp.transpose(new_xyz, (0, 2, 1))
    new_pts = jnp.transpose(feats.reshape(B, s, -1), (0, 2, 1))
    return new_l_xyz, new_pts


def _fp_level(xyz1, xyz2, pts1, pts2, ws, head=None):
    n1 = xyz1.shape[2]
    n2 = xyz2.shape[2]
    c2 = pts2.shape[1]
    w3, idx3 = _top3_call(jnp.transpose(xyz1, (0, 2, 1)), xyz2)
    table = _pad_cols(jnp.transpose(pts2, (0, 2, 1)).reshape(B * n2, c2))
    g = _sc_gather(table, idx3.reshape(-1))
    p1_rows = None
    if pts1 is not None:
        p1_rows = jnp.transpose(pts1, (0, 2, 1)).reshape(B * n1, -1)
    out = _fpmlp_call(g, w3.reshape(B * n1 * 3, 1), p1_rows, ws, head)
    return out, n1


def kernel(xyz, input_for_alignment_all_structure, params):
    xyz = xyz.astype(jnp.float32)
    n = xyz.shape[2]
    trans, l0_points = _stn_call(xyz, params)
    l0_xyz = xyz[:, :3, :]

    l1_xyz, l1_points = _sa_level(l0_xyz, l0_points, 1024, 0.1,
                                  _mlp_params(params, 'sa1', 3))
    l2_xyz, l2_points = _sa_level(l1_xyz, l1_points, 256, 0.2,
                                  _mlp_params(params, 'sa2', 3))
    l3_xyz, l3_points = _sa_level(l2_xyz, l2_points, 64, 0.4,
                                  _mlp_params(params, 'sa3', 3))
    l4_xyz, l4_points = _sa_level(l3_xyz, l3_points, 16, 0.8,
                                  _mlp_params(params, 'sa4', 3))

    o, n1 = _fp_level(l3_xyz, l4_xyz, l3_points, l4_points,
                      _mlp_params(params, 'fp4', 2))
    fp4_pts = jnp.transpose(o.reshape(B, n1, -1), (0, 2, 1))
    o, n1 = _fp_level(l2_xyz, l3_xyz, l2_points, fp4_pts,
                      _mlp_params(params, 'fp3', 2))
    fp3_pts = jnp.transpose(o.reshape(B, n1, -1), (0, 2, 1))
    o, n1 = _fp_level(l1_xyz, l2_xyz, l1_points, fp3_pts,
                      _mlp_params(params, 'fp2', 2))
    fp2_pts = jnp.transpose(o.reshape(B, n1, -1), (0, 2, 1))
    head = (params['head_conv1_w'], params['head_conv1_b'],
            params['head_conv2_w'], params['head_conv2_b'])
    o, n1 = _fp_level(l0_xyz, l1_xyz, None, fp2_pts,
                      _mlp_params(params, 'fp1', 3), head=head)
    x = o.reshape(B, n, 7)
    return (x, trans)
